# baseline (speedup 1.0000x reference)
import functools

import jax
import jax.numpy as jnp
from jax.experimental import pallas as pl

TOPK = 32
NRBF = 16
MAXREL = 32

_OFF = {"N": 0, "Ca": 3, "C": 6, "O": 9, "Cb": 12}
_PAIRS = [("N", "N"), ("C", "C"), ("O", "O"), ("Cb", "Cb"), ("Ca", "N"),
          ("Ca", "C"), ("Ca", "O"), ("Ca", "Cb"), ("N", "C"), ("N", "O"),
          ("N", "Cb"), ("Cb", "C"), ("Cb", "O"), ("O", "C"), ("N", "Ca"),
          ("C", "Ca"), ("O", "Ca"), ("Cb", "Ca"), ("C", "N"), ("O", "N"),
          ("Cb", "N"), ("C", "Cb"), ("O", "Cb"), ("C", "O")]


def _topk_feat_kernel(x_ref, cac_ref, mrow_ref, mcol_ref, rid_ref, ch_ref,
                      eidx_ref, dnb_ref, f_ref, *, R, N, K):
    x = x_ref[0]
    n_at = x[:, 0:3]
    ca = x[:, 3:6]
    c_at = x[:, 6:9]
    o_at = x[:, 9:12]
    bv = ca - n_at
    cv = c_at - ca
    a0 = bv[:, 1:2] * cv[:, 2:3] - bv[:, 2:3] * cv[:, 1:2]
    a1 = bv[:, 2:3] * cv[:, 0:1] - bv[:, 0:1] * cv[:, 2:3]
    a2 = bv[:, 0:1] * cv[:, 1:2] - bv[:, 1:2] * cv[:, 0:1]
    av = jnp.concatenate([a0, a1, a2], axis=1)
    cb = -0.58273431 * av + 0.56802827 * bv - 0.54067466 * cv + ca
    f_ref[0] = jnp.concatenate(
        [n_at, ca, c_at, o_at, cb, rid_ref[0], ch_ref[0]], axis=1)

    cac = cac_ref[0]
    acc = jnp.zeros((R, N), jnp.float32)
    for d in range(3):
        diff = ca[:, d:d + 1] - cac[d:d + 1, :]
        acc = acc + diff * diff
    m2 = mrow_ref[0] * mcol_ref[0]
    D = m2 * jnp.sqrt(acc + 1e-6)
    dmax = jnp.max(D, axis=1, keepdims=True)
    Dadj = D + (1.0 - m2) * dmax

    lane = jax.lax.broadcasted_iota(jnp.int32, (R, N), 1)
    idx_cols = []
    val_cols = []
    for _ in range(K):
        m = jnp.min(Dadj, axis=1, keepdims=True)
        idx = jnp.min(jnp.where(Dadj == m, lane, N), axis=1, keepdims=True)
        idx_cols.append(idx)
        val_cols.append(m)
        Dadj = jnp.where(lane == idx, 1e30, Dadj)
    eidx_ref[0] = jnp.concatenate(idx_cols, axis=1)
    dnb_ref[0] = jnp.concatenate(val_cols, axis=1)


def _edge_feat_kernel(eidx_ref, dnb_ref, f_ref, wpos_ref, bpos_ref,
                      wedge_ref, g_ref, b_ref, out_ref, *, R, N, K):
    E = R * K
    e_i = eidx_ref[0]
    d0 = dnb_ref[0]
    fall = f_ref[0]

    oh = (e_i == jax.lax.broadcasted_iota(jnp.int32, (E, N), 1)
          ).astype(jnp.float32)
    G = jnp.dot(oh, fall, preferred_element_type=jnp.float32,
                precision=jax.lax.Precision.HIGHEST)
    i = pl.program_id(1)
    fi = f_ref[0, pl.ds(i * R, R), :]
    Fi = jnp.broadcast_to(fi[:, None, :], (R, K, 17)).reshape(E, 17)

    offs = Fi[:, 15:16] - G[:, 15:16]
    ceq = (Fi[:, 16:17] == G[:, 16:17]).astype(jnp.float32)
    dpos = (jnp.clip(offs + float(MAXREL), 0.0, float(2 * MAXREL)) * ceq
            + (1.0 - ceq) * float(2 * MAXREL + 1))
    pos_iota = jax.lax.broadcasted_iota(jnp.int32, (E, 2 * MAXREL + 2), 1)
    oh66 = (dpos.astype(jnp.int32) == pos_iota).astype(jnp.float32)
    epos = jnp.dot(oh66, wpos_ref[...], preferred_element_type=jnp.float32,
                   precision=jax.lax.Precision.HIGHEST) + bpos_ref[...]

    mu = 2.0 + jax.lax.broadcasted_iota(
        jnp.int32, (1, NRBF), 1).astype(jnp.float32) * (20.0 / 15.0)
    sigma = 20.0 / NRBF

    def rbf(dcol):
        z = (dcol - mu) / sigma
        return jnp.exp(-(z * z))

    chunks = [epos, rbf(d0)]
    for a_name, b_name in _PAIRS:
        sa, sb = _OFF[a_name], _OFF[b_name]
        da = Fi[:, sa:sa + 3] - G[:, sb:sb + 3]
        d2 = jnp.sum(da * da, axis=1, keepdims=True)
        chunks.append(rbf(jnp.sqrt(d2 + 1e-6)))
    feat = jnp.concatenate(chunks, axis=1)

    out = jnp.dot(feat, wedge_ref[...], preferred_element_type=jnp.float32)
    mu_o = jnp.mean(out, axis=1, keepdims=True)
    var = jnp.mean((out - mu_o) ** 2, axis=1, keepdims=True)
    out = (out - mu_o) * jax.lax.rsqrt(var + 1e-5) * g_ref[...] + b_ref[...]
    out_ref[0] = out


@jax.jit
def kernel(X, mask, residue_idx, chain_labels, W_pos, b_pos, W_edge, ln_g, ln_b):
    B, N = X.shape[0], X.shape[1]
    K = min(TOPK, N)
    R = 256
    R2 = 32
    E = R2 * K

    x_rows = X.reshape(B, N, 12)
    ca_cols = X[:, :, 1, :].transpose(0, 2, 1)
    mrow = mask[:, :, None]
    mcol = mask[:, None, :]
    rid = residue_idx.astype(jnp.float32)[:, :, None]
    ch = chain_labels.astype(jnp.float32)[:, :, None]

    eidx, dnb, F = pl.pallas_call(
        functools.partial(_topk_feat_kernel, R=R, N=N, K=K),
        grid=(B, N // R),
        in_specs=[
            pl.BlockSpec((1, R, 12), lambda b, i: (b, i, 0)),
            pl.BlockSpec((1, 3, N), lambda b, i: (b, 0, 0)),
            pl.BlockSpec((1, R, 1), lambda b, i: (b, i, 0)),
            pl.BlockSpec((1, 1, N), lambda b, i: (b, 0, 0)),
            pl.BlockSpec((1, R, 1), lambda b, i: (b, i, 0)),
            pl.BlockSpec((1, R, 1), lambda b, i: (b, i, 0)),
        ],
        out_specs=[
            pl.BlockSpec((1, R, K), lambda b, i: (b, i, 0)),
            pl.BlockSpec((1, R, K), lambda b, i: (b, i, 0)),
            pl.BlockSpec((1, R, 17), lambda b, i: (b, i, 0)),
        ],
        out_shape=[
            jax.ShapeDtypeStruct((B, N, K), jnp.int32),
            jax.ShapeDtypeStruct((B, N, K), jnp.float32),
            jax.ShapeDtypeStruct((B, N, 17), jnp.float32),
        ],
    )(x_rows, ca_cols, mrow, mcol, rid, ch)

    eidx_flat = eidx.reshape(B, N * K, 1)
    dnb_flat = dnb.reshape(B, N * K, 1)

    out = pl.pallas_call(
        functools.partial(_edge_feat_kernel, R=R2, N=N, K=K),
        grid=(B, N // R2),
        in_specs=[
            pl.BlockSpec((1, E, 1), lambda b, i: (b, i, 0)),
            pl.BlockSpec((1, E, 1), lambda b, i: (b, i, 0)),
            pl.BlockSpec((1, N, 17), lambda b, i: (b, 0, 0)),
            pl.BlockSpec((2 * MAXREL + 2, NRBF), lambda b, i: (0, 0)),
            pl.BlockSpec((1, NRBF), lambda b, i: (0, 0)),
            pl.BlockSpec((416, 128), lambda b, i: (0, 0)),
            pl.BlockSpec((1, 128), lambda b, i: (0, 0)),
            pl.BlockSpec((1, 128), lambda b, i: (0, 0)),
        ],
        out_specs=pl.BlockSpec((1, E, 128), lambda b, i: (b, i, 0)),
        out_shape=jax.ShapeDtypeStruct((B, N * K, 128), jnp.float32),
    )(eidx_flat, dnb_flat, F, W_pos.T, b_pos[None, :], W_edge.T,
      ln_g[None, :], ln_b[None, :])

    return out.reshape(B, N, K, 128), eidx


# speedup vs baseline: 1.1986x; 1.1986x over previous
import functools

import jax
import jax.numpy as jnp
import numpy as np
from jax.experimental import pallas as pl

TOPK = 32
NRBF = 16
MAXREL = 32

_OFF = {"N": 0, "Ca": 3, "C": 6, "O": 9, "Cb": 12}
_PAIRS = [("N", "N"), ("C", "C"), ("O", "O"), ("Cb", "Cb"), ("Ca", "N"),
          ("Ca", "C"), ("Ca", "O"), ("Ca", "Cb"), ("N", "C"), ("N", "O"),
          ("N", "Cb"), ("Cb", "C"), ("Cb", "O"), ("O", "C"), ("N", "Ca"),
          ("C", "Ca"), ("O", "Ca"), ("Cb", "Ca"), ("C", "N"), ("O", "N"),
          ("Cb", "N"), ("C", "Cb"), ("O", "Cb"), ("C", "O")]

_SEL_A = np.zeros((17, 72), np.float32)
_SEL_B = np.zeros((17, 72), np.float32)
_SUM3 = np.zeros((72, 24), np.float32)
for _p, (_a, _b) in enumerate(_PAIRS):
    for _d in range(3):
        _SEL_A[_OFF[_a] + _d, _p * 3 + _d] = 1.0
        _SEL_B[_OFF[_b] + _d, _p * 3 + _d] = 1.0
        _SUM3[_p * 3 + _d, _p] = 1.0
_REP16 = np.zeros((25, 400), np.float32)
for _p in range(25):
    _REP16[_p, _p * 16:(_p + 1) * 16] = 1.0


def _topk_feat_kernel(x_ref, cac_ref, mrow_ref, mcol_ref, rid_ref, ch_ref,
                      eidx_ref, dnb_ref, f_ref, *, R, N, K):
    x = x_ref[0]
    n_at = x[:, 0:3]
    ca = x[:, 3:6]
    c_at = x[:, 6:9]
    o_at = x[:, 9:12]
    bv = ca - n_at
    cv = c_at - ca
    a0 = bv[:, 1:2] * cv[:, 2:3] - bv[:, 2:3] * cv[:, 1:2]
    a1 = bv[:, 2:3] * cv[:, 0:1] - bv[:, 0:1] * cv[:, 2:3]
    a2 = bv[:, 0:1] * cv[:, 1:2] - bv[:, 1:2] * cv[:, 0:1]
    av = jnp.concatenate([a0, a1, a2], axis=1)
    cb = -0.58273431 * av + 0.56802827 * bv - 0.54067466 * cv + ca
    f_ref[0] = jnp.concatenate(
        [n_at, ca, c_at, o_at, cb, rid_ref[0], ch_ref[0]], axis=1)

    cac = cac_ref[0]
    acc = jnp.zeros((R, N), jnp.float32)
    for d in range(3):
        diff = ca[:, d:d + 1] - cac[d:d + 1, :]
        acc = acc + diff * diff
    m2 = mrow_ref[0] * mcol_ref[0]
    D = m2 * jnp.sqrt(acc + 1e-6)
    dmax = jnp.max(D, axis=1, keepdims=True)
    Dadj = D + (1.0 - m2) * dmax

    lane = jax.lax.broadcasted_iota(jnp.int32, (R, N), 1)
    idx_cols = []
    val_cols = []
    for _ in range(K):
        m = jnp.min(Dadj, axis=1, keepdims=True)
        idx = jnp.min(jnp.where(Dadj == m, lane, N), axis=1, keepdims=True)
        idx_cols.append(idx)
        val_cols.append(m)
        Dadj = jnp.where(lane == idx, 1e30, Dadj)
    eidx_ref[0] = jnp.concatenate(idx_cols, axis=1)
    dnb_ref[0] = jnp.concatenate(val_cols, axis=1)


def _edge_feat_kernel(eidx_ref, dnb_ref, f_ref, wpos_ref, bpos_ref,
                      wedge_ref, g_ref, b_ref, sa_ref, sb_ref, s3_ref,
                      rep_ref, out_ref, *, R, N, K):
    E = R * K
    HIGH = jax.lax.Precision.HIGHEST
    e_i = eidx_ref[0]
    d0 = dnb_ref[0]
    fall = f_ref[0]

    oh = (e_i == jax.lax.broadcasted_iota(jnp.int32, (E, N), 1)
          ).astype(jnp.float32)
    G = jnp.dot(oh, fall, preferred_element_type=jnp.float32,
                precision=HIGH)
    i = pl.program_id(1)
    fi = f_ref[0, pl.ds(i * R, R), :]
    Fi = jnp.broadcast_to(fi[:, None, :], (R, K, 17)).reshape(E, 17)

    offs = Fi[:, 15:16] - G[:, 15:16]
    ceq = (Fi[:, 16:17] == G[:, 16:17]).astype(jnp.float32)
    dpos = (jnp.clip(offs + float(MAXREL), 0.0, float(2 * MAXREL)) * ceq
            + (1.0 - ceq) * float(2 * MAXREL + 1))
    pos_iota = jax.lax.broadcasted_iota(jnp.int32, (E, 2 * MAXREL + 2), 1)
    oh66 = (dpos.astype(jnp.int32) == pos_iota).astype(jnp.float32)
    epos = jnp.dot(oh66, wpos_ref[...], preferred_element_type=jnp.float32,
                   precision=HIGH) + bpos_ref[...]

    AD = (jnp.dot(Fi, sa_ref[...], preferred_element_type=jnp.float32,
                  precision=HIGH)
          - jnp.dot(G, sb_ref[...], preferred_element_type=jnp.float32,
                    precision=HIGH))
    d2 = jnp.dot(AD * AD, s3_ref[...], preferred_element_type=jnp.float32,
                 precision=HIGH)
    dpair = jnp.sqrt(d2 + 1e-6)
    dall = jnp.concatenate([d0, dpair], axis=1)
    dexp = jnp.dot(dall, rep_ref[...], preferred_element_type=jnp.float32,
                   precision=HIGH)
    lane = jax.lax.broadcasted_iota(jnp.int32, (1, 25 * NRBF), 1)
    mu = 2.0 + jnp.bitwise_and(lane, NRBF - 1).astype(jnp.float32) * (20.0 / 15.0)
    z = (dexp - mu) * (1.0 / (20.0 / NRBF))
    rbfs = jnp.exp(-(z * z))
    feat = jnp.concatenate([epos, rbfs], axis=1)

    out = jnp.dot(feat, wedge_ref[...], preferred_element_type=jnp.float32)
    mu_o = jnp.mean(out, axis=1, keepdims=True)
    var = jnp.mean((out - mu_o) ** 2, axis=1, keepdims=True)
    out = (out - mu_o) * jax.lax.rsqrt(var + 1e-5) * g_ref[...] + b_ref[...]
    out_ref[0] = out


@jax.jit
def kernel(X, mask, residue_idx, chain_labels, W_pos, b_pos, W_edge, ln_g, ln_b):
    B, N = X.shape[0], X.shape[1]
    K = min(TOPK, N)
    R = 256
    R2 = 32
    E = R2 * K

    x_rows = X.reshape(B, N, 12)
    ca_cols = X[:, :, 1, :].transpose(0, 2, 1)
    mrow = mask[:, :, None]
    mcol = mask[:, None, :]
    rid = residue_idx.astype(jnp.float32)[:, :, None]
    ch = chain_labels.astype(jnp.float32)[:, :, None]

    eidx, dnb, F = pl.pallas_call(
        functools.partial(_topk_feat_kernel, R=R, N=N, K=K),
        grid=(B, N // R),
        in_specs=[
            pl.BlockSpec((1, R, 12), lambda b, i: (b, i, 0)),
            pl.BlockSpec((1, 3, N), lambda b, i: (b, 0, 0)),
            pl.BlockSpec((1, R, 1), lambda b, i: (b, i, 0)),
            pl.BlockSpec((1, 1, N), lambda b, i: (b, 0, 0)),
            pl.BlockSpec((1, R, 1), lambda b, i: (b, i, 0)),
            pl.BlockSpec((1, R, 1), lambda b, i: (b, i, 0)),
        ],
        out_specs=[
            pl.BlockSpec((1, R, K), lambda b, i: (b, i, 0)),
            pl.BlockSpec((1, R, K), lambda b, i: (b, i, 0)),
            pl.BlockSpec((1, R, 17), lambda b, i: (b, i, 0)),
        ],
        out_shape=[
            jax.ShapeDtypeStruct((B, N, K), jnp.int32),
            jax.ShapeDtypeStruct((B, N, K), jnp.float32),
            jax.ShapeDtypeStruct((B, N, 17), jnp.float32),
        ],
    )(x_rows, ca_cols, mrow, mcol, rid, ch)

    eidx_flat = eidx.reshape(B, N * K, 1)
    dnb_flat = dnb.reshape(B, N * K, 1)

    out = pl.pallas_call(
        functools.partial(_edge_feat_kernel, R=R2, N=N, K=K),
        grid=(B, N // R2),
        in_specs=[
            pl.BlockSpec((1, E, 1), lambda b, i: (b, i, 0)),
            pl.BlockSpec((1, E, 1), lambda b, i: (b, i, 0)),
            pl.BlockSpec((1, N, 17), lambda b, i: (b, 0, 0)),
            pl.BlockSpec((2 * MAXREL + 2, NRBF), lambda b, i: (0, 0)),
            pl.BlockSpec((1, NRBF), lambda b, i: (0, 0)),
            pl.BlockSpec((416, 128), lambda b, i: (0, 0)),
            pl.BlockSpec((1, 128), lambda b, i: (0, 0)),
            pl.BlockSpec((1, 128), lambda b, i: (0, 0)),
            pl.BlockSpec((17, 72), lambda b, i: (0, 0)),
            pl.BlockSpec((17, 72), lambda b, i: (0, 0)),
            pl.BlockSpec((72, 24), lambda b, i: (0, 0)),
            pl.BlockSpec((25, 400), lambda b, i: (0, 0)),
        ],
        out_specs=pl.BlockSpec((1, E, 128), lambda b, i: (b, i, 0)),
        out_shape=jax.ShapeDtypeStruct((B, N * K, 128), jnp.float32),
    )(eidx_flat, dnb_flat, F, W_pos.T, b_pos[None, :], W_edge.T,
      ln_g[None, :], ln_b[None, :], jnp.asarray(_SEL_A), jnp.asarray(_SEL_B),
      jnp.asarray(_SUM3), jnp.asarray(_REP16))

    return out.reshape(B, N, K, 128), eidx


# speedup vs baseline: 1.8303x; 1.5271x over previous
import functools

import jax
import jax.numpy as jnp
import numpy as np
from jax.experimental import pallas as pl

TOPK = 32
NRBF = 16
MAXREL = 32

_OFF = {"N": 0, "Ca": 3, "C": 6, "O": 9, "Cb": 12}
_PAIRS = [("N", "N"), ("C", "C"), ("O", "O"), ("Cb", "Cb"), ("Ca", "N"),
          ("Ca", "C"), ("Ca", "O"), ("Ca", "Cb"), ("N", "C"), ("N", "O"),
          ("N", "Cb"), ("Cb", "C"), ("Cb", "O"), ("O", "C"), ("N", "Ca"),
          ("C", "Ca"), ("O", "Ca"), ("Cb", "Ca"), ("C", "N"), ("O", "N"),
          ("Cb", "N"), ("C", "Cb"), ("O", "Cb"), ("C", "O")]

_SEL_A = np.zeros((17, 72), np.float32)
_SEL_B = np.zeros((17, 72), np.float32)
_SUM3 = np.zeros((72, 24), np.float32)
for _p, (_a, _b) in enumerate(_PAIRS):
    for _d in range(3):
        _SEL_A[_OFF[_a] + _d, _p * 3 + _d] = 1.0
        _SEL_B[_OFF[_b] + _d, _p * 3 + _d] = 1.0
        _SUM3[_p * 3 + _d, _p] = 1.0
_REP16 = np.zeros((25, 400), np.float32)
for _p in range(25):
    _REP16[_p, _p * 16:(_p + 1) * 16] = 1.0


def _topk_feat_kernel(x_ref, cac_ref, mrow_ref, mcol_ref, rid_ref, ch_ref,
                      eidx_ref, dnb_ref, f_ref, *, R, N, K):
    x = x_ref[0]
    n_at = x[:, 0:3]
    ca = x[:, 3:6]
    c_at = x[:, 6:9]
    o_at = x[:, 9:12]
    bv = ca - n_at
    cv = c_at - ca
    a0 = bv[:, 1:2] * cv[:, 2:3] - bv[:, 2:3] * cv[:, 1:2]
    a1 = bv[:, 2:3] * cv[:, 0:1] - bv[:, 0:1] * cv[:, 2:3]
    a2 = bv[:, 0:1] * cv[:, 1:2] - bv[:, 1:2] * cv[:, 0:1]
    av = jnp.concatenate([a0, a1, a2], axis=1)
    cb = -0.58273431 * av + 0.56802827 * bv - 0.54067466 * cv + ca
    f_ref[0] = jnp.concatenate(
        [n_at, ca, c_at, o_at, cb, rid_ref[0], ch_ref[0]], axis=1)

    cac = cac_ref[0]
    acc = jnp.zeros((R, N), jnp.float32)
    for d in range(3):
        diff = ca[:, d:d + 1] - cac[d:d + 1, :]
        acc = acc + diff * diff
    m2 = mrow_ref[0] * mcol_ref[0]
    D = m2 * jnp.sqrt(acc + 1e-6)
    dmax = jnp.max(D, axis=1, keepdims=True)
    Dadj = D + (1.0 - m2) * dmax

    lane = jax.lax.broadcasted_iota(jnp.int32, (R, N), 1)
    idx_cols = []
    val_cols = []
    for _ in range(K):
        m = jnp.min(Dadj, axis=1, keepdims=True)
        idx = jnp.min(jnp.where(Dadj == m, lane, N), axis=1, keepdims=True)
        idx_cols.append(idx)
        val_cols.append(m)
        Dadj = jnp.where(lane == idx, 1e30, Dadj)
    eidx_ref[0] = jnp.concatenate(idx_cols, axis=1)
    dnb_ref[0] = jnp.concatenate(val_cols, axis=1)


def _edge_feat_kernel(eidx_ref, dnb_ref, f_ref, wpos_ref, bpos_ref,
                      wedge_ref, g_ref, b_ref, sa_ref, sb_ref, s3_ref,
                      rep_ref, out_ref, *, R, N, K):
    E = R * K
    HIGH = jax.lax.Precision.HIGHEST
    e_i = eidx_ref[0]
    d0 = dnb_ref[0]
    fall = f_ref[0]

    oh = (e_i == jax.lax.broadcasted_iota(jnp.int32, (E, N), 1)
          ).astype(jnp.bfloat16)
    fall_hi = fall.astype(jnp.bfloat16)
    fall_lo = (fall - fall_hi.astype(jnp.float32)).astype(jnp.bfloat16)
    G = (jnp.dot(oh, fall_hi, preferred_element_type=jnp.float32)
         + jnp.dot(oh, fall_lo, preferred_element_type=jnp.float32))
    i = pl.program_id(1)
    fi = f_ref[0, pl.ds(i * R, R), :]
    Fi = jnp.broadcast_to(fi[:, None, :], (R, K, 17)).reshape(E, 17)

    offs = Fi[:, 15:16] - G[:, 15:16]
    ceq = (Fi[:, 16:17] == G[:, 16:17]).astype(jnp.float32)
    dpos = (jnp.clip(offs + float(MAXREL), 0.0, float(2 * MAXREL)) * ceq
            + (1.0 - ceq) * float(2 * MAXREL + 1))
    pos_iota = jax.lax.broadcasted_iota(jnp.int32, (E, 2 * MAXREL + 2), 1)
    oh66 = (dpos.astype(jnp.int32) == pos_iota).astype(jnp.float32)
    epos = jnp.dot(oh66, wpos_ref[...], preferred_element_type=jnp.float32,
                   precision=HIGH) + bpos_ref[...]

    AD = (jnp.dot(Fi, sa_ref[...], preferred_element_type=jnp.float32,
                  precision=HIGH)
          - jnp.dot(G, sb_ref[...], preferred_element_type=jnp.float32,
                    precision=HIGH))
    d2 = jnp.dot(AD * AD, s3_ref[...], preferred_element_type=jnp.float32,
                 precision=HIGH)
    dpair = jnp.sqrt(d2 + 1e-6)
    dall = jnp.concatenate([d0, dpair], axis=1)
    rep_b = rep_ref[...].astype(jnp.bfloat16)
    dall_hi = dall.astype(jnp.bfloat16)
    dall_lo = (dall - dall_hi.astype(jnp.float32)).astype(jnp.bfloat16)
    dexp = (jnp.dot(dall_hi, rep_b, preferred_element_type=jnp.float32)
            + jnp.dot(dall_lo, rep_b, preferred_element_type=jnp.float32))
    lane = jax.lax.broadcasted_iota(jnp.int32, (1, 25 * NRBF), 1)
    mu = 2.0 + jnp.bitwise_and(lane, NRBF - 1).astype(jnp.float32) * (20.0 / 15.0)
    z = (dexp - mu) * (1.0 / (20.0 / NRBF))
    rbfs = jnp.exp(-(z * z))
    feat = jnp.concatenate([epos, rbfs], axis=1)

    out = jnp.dot(feat, wedge_ref[...], preferred_element_type=jnp.float32)
    mu_o = jnp.mean(out, axis=1, keepdims=True)
    var = jnp.mean((out - mu_o) ** 2, axis=1, keepdims=True)
    out = (out - mu_o) * jax.lax.rsqrt(var + 1e-5) * g_ref[...] + b_ref[...]
    out_ref[0] = out


@jax.jit
def kernel(X, mask, residue_idx, chain_labels, W_pos, b_pos, W_edge, ln_g, ln_b):
    B, N = X.shape[0], X.shape[1]
    K = min(TOPK, N)
    R = 256
    R2 = 64
    E = R2 * K

    x_rows = X.reshape(B, N, 12)
    ca_cols = X[:, :, 1, :].transpose(0, 2, 1)
    mrow = mask[:, :, None]
    mcol = mask[:, None, :]
    rid = residue_idx.astype(jnp.float32)[:, :, None]
    ch = chain_labels.astype(jnp.float32)[:, :, None]

    eidx, dnb, F = pl.pallas_call(
        functools.partial(_topk_feat_kernel, R=R, N=N, K=K),
        grid=(B, N // R),
        in_specs=[
            pl.BlockSpec((1, R, 12), lambda b, i: (b, i, 0)),
            pl.BlockSpec((1, 3, N), lambda b, i: (b, 0, 0)),
            pl.BlockSpec((1, R, 1), lambda b, i: (b, i, 0)),
            pl.BlockSpec((1, 1, N), lambda b, i: (b, 0, 0)),
            pl.BlockSpec((1, R, 1), lambda b, i: (b, i, 0)),
            pl.BlockSpec((1, R, 1), lambda b, i: (b, i, 0)),
        ],
        out_specs=[
            pl.BlockSpec((1, R, K), lambda b, i: (b, i, 0)),
            pl.BlockSpec((1, R, K), lambda b, i: (b, i, 0)),
            pl.BlockSpec((1, R, 17), lambda b, i: (b, i, 0)),
        ],
        out_shape=[
            jax.ShapeDtypeStruct((B, N, K), jnp.int32),
            jax.ShapeDtypeStruct((B, N, K), jnp.float32),
            jax.ShapeDtypeStruct((B, N, 17), jnp.float32),
        ],
    )(x_rows, ca_cols, mrow, mcol, rid, ch)

    eidx_flat = eidx.reshape(B, N * K, 1)
    dnb_flat = dnb.reshape(B, N * K, 1)

    out = pl.pallas_call(
        functools.partial(_edge_feat_kernel, R=R2, N=N, K=K),
        grid=(B, N // R2),
        in_specs=[
            pl.BlockSpec((1, E, 1), lambda b, i: (b, i, 0)),
            pl.BlockSpec((1, E, 1), lambda b, i: (b, i, 0)),
            pl.BlockSpec((1, N, 17), lambda b, i: (b, 0, 0)),
            pl.BlockSpec((2 * MAXREL + 2, NRBF), lambda b, i: (0, 0)),
            pl.BlockSpec((1, NRBF), lambda b, i: (0, 0)),
            pl.BlockSpec((416, 128), lambda b, i: (0, 0)),
            pl.BlockSpec((1, 128), lambda b, i: (0, 0)),
            pl.BlockSpec((1, 128), lambda b, i: (0, 0)),
            pl.BlockSpec((17, 72), lambda b, i: (0, 0)),
            pl.BlockSpec((17, 72), lambda b, i: (0, 0)),
            pl.BlockSpec((72, 24), lambda b, i: (0, 0)),
            pl.BlockSpec((25, 400), lambda b, i: (0, 0)),
        ],
        out_specs=pl.BlockSpec((1, E, 128), lambda b, i: (b, i, 0)),
        out_shape=jax.ShapeDtypeStruct((B, N * K, 128), jnp.float32),
    )(eidx_flat, dnb_flat, F, W_pos.T, b_pos[None, :], W_edge.T,
      ln_g[None, :], ln_b[None, :], jnp.asarray(_SEL_A), jnp.asarray(_SEL_B),
      jnp.asarray(_SUM3), jnp.asarray(_REP16))

    return out.reshape(B, N, K, 128), eidx
